# baseline (speedup 1.0000x reference)
import functools

import jax
import jax.numpy as jnp
from jax import lax
from jax.experimental import pallas as pl
from jax.experimental.pallas import tpu as pltpu
from jax.experimental.pallas import tpu_sc as plsc

NC = 2
NS = 16
L = 16
NW = NC * NS

TOKENS = 8192
D = 768
R = TOKENS // NW
C = 32
NCH = R // C
DL = D // L

_mesh = plsc.VectorSubcoreMesh(core_axis_name="c", subcore_axis_name="s")


@functools.partial(
    pl.kernel,
    out_type=(
        jax.ShapeDtypeStruct((TOKENS, D), jnp.float32),
        jax.ShapeDtypeStruct((TOKENS,), jnp.float32),
    ),
    mesh=_mesh,
    scratch_types=[
        pltpu.VMEM((R,), jnp.int32),
        pltpu.VMEM((R,), jnp.int32),
        pltpu.VMEM((2, C, D), jnp.float32),
        pltpu.VMEM((2, C, D), jnp.float32),
        pltpu.VMEM((R,), jnp.float32),
        pltpu.SemaphoreType.DMA,
        pltpu.SemaphoreType.DMA,
        pltpu.SemaphoreType.DMA,
        pltpu.SemaphoreType.DMA,
    ],
)
def _emb_kernel(ids_hbm, pos_hbm, am_hbm, wte_hbm, wpe_hbm, out_hbm, mask_hbm,
                tok_idx, pos_idx, tok_rows, pos_rows, am_v,
                sem_g0, sem_g1, sem_o0, sem_o1):
    wid = lax.axis_index("s") * NC + lax.axis_index("c")
    base = wid * R
    sem_g = (sem_g0, sem_g1)
    sem_o = (sem_o0, sem_o1)

    pltpu.sync_copy(ids_hbm.at[pl.ds(base, R)], tok_idx)
    pltpu.sync_copy(pos_hbm.at[pl.ds(base, R)], pos_idx)

    def start_gather(i, b):
        sl = pl.ds(i * C, C)
        ht = pltpu.async_copy(wte_hbm.at[tok_idx.at[sl]], tok_rows.at[b], sem_g[b])
        hp = pltpu.async_copy(wpe_hbm.at[pos_idx.at[sl]], pos_rows.at[b], sem_g[b])
        return (ht, hp)

    gh = [None, None]
    oh = [None, None]
    gh[0] = start_gather(0, 0)

    pltpu.sync_copy(am_hbm.at[pl.ds(base, R)], am_v)

    @pl.loop(0, R // L)
    def _mask(j):
        s = pl.ds(j * L, L)
        am_v[s] = (am_v[s] - 1.0) * 10000.0

    pltpu.sync_copy(am_v, mask_hbm.at[pl.ds(base, R)])

    for i in range(NCH):
        b = i % 2
        nb = 1 - b
        if i + 1 < NCH:
            if oh[nb] is not None:
                oh[nb].wait()
                oh[nb] = None
            gh[nb] = start_gather(i + 1, nb)
        gh[b][0].wait()
        gh[b][1].wait()

        @pl.loop(0, C)
        def _row(r):
            for j in range(DL):
                s = pl.ds(j * L, L)
                tok_rows[b, r, s] = tok_rows[b, r, s] + pos_rows[b, r, s]

        oh[b] = pltpu.async_copy(
            tok_rows.at[b], out_hbm.at[pl.ds(base + i * C, C)], sem_o[b])

    for h in oh:
        if h is not None:
            h.wait()


def kernel(input_ids, attention_mask, position_ids, wte, wpe):
    B, S = input_ids.shape
    ids = input_ids.reshape(-1).astype(jnp.int32)
    pos = position_ids.reshape(-1).astype(jnp.int32)
    am = attention_mask.reshape(-1)
    hidden, mask = _emb_kernel(ids, pos, am, wte, wpe)
    return (hidden.reshape(B, S, D), mask.reshape(1, 1, B, S))


# speedup vs baseline: 1.0668x; 1.0668x over previous
import functools

import jax
import jax.numpy as jnp
from jax import lax
from jax.experimental import pallas as pl
from jax.experimental.pallas import tpu as pltpu
from jax.experimental.pallas import tpu_sc as plsc

NC = 2
NS = 16
L = 16
NW = NC * NS

TOKENS = 8192
D = 768
R = TOKENS // NW
C = 32
NCH = R // C
DL = D // L

_mesh = plsc.VectorSubcoreMesh(core_axis_name="c", subcore_axis_name="s")


@functools.partial(
    pl.kernel,
    out_type=(
        jax.ShapeDtypeStruct((TOKENS, D), jnp.float32),
        jax.ShapeDtypeStruct((TOKENS,), jnp.float32),
    ),
    mesh=_mesh,
    scratch_types=[
        pltpu.VMEM((R,), jnp.int32),
        pltpu.VMEM((R,), jnp.int32),
        pltpu.VMEM((2, C, D), jnp.float32),
        pltpu.VMEM((2, C, D), jnp.float32),
        pltpu.VMEM((R,), jnp.float32),
        pltpu.SemaphoreType.DMA,
        pltpu.SemaphoreType.DMA,
        pltpu.SemaphoreType.DMA,
        pltpu.SemaphoreType.DMA,
    ],
)
def _emb_kernel(ids_hbm, pos_hbm, am_hbm, wte_hbm, wpe_hbm, out_hbm, mask_hbm,
                tok_idx, pos_idx, tok_rows, pos_rows, am_v,
                sem_g0, sem_g1, sem_o0, sem_o1):
    wid = lax.axis_index("s") * NC + lax.axis_index("c")
    base = wid * R
    sem_g = (sem_g0, sem_g1)
    sem_o = (sem_o0, sem_o1)

    pltpu.sync_copy(ids_hbm.at[pl.ds(base, R)], tok_idx)
    pltpu.sync_copy(pos_hbm.at[pl.ds(base, R)], pos_idx)

    def start_gather(g, b):
        sl = pl.ds(g * C, C)
        pltpu.async_copy(wte_hbm.at[tok_idx.at[sl]], tok_rows.at[b], sem_g[b])
        pltpu.async_copy(wpe_hbm.at[pos_idx.at[sl]], pos_rows.at[b], sem_g[b])

    def wait_gather(b):
        pltpu.make_async_copy(
            wte_hbm.at[pl.ds(0, C)], tok_rows.at[b], sem_g[b]).wait()
        pltpu.make_async_copy(
            wpe_hbm.at[pl.ds(0, C)], pos_rows.at[b], sem_g[b]).wait()

    def wait_out(b):
        pltpu.make_async_copy(
            tok_rows.at[b], out_hbm.at[pl.ds(0, C)], sem_o[b]).wait()

    start_gather(0, 0)

    pltpu.sync_copy(am_hbm.at[pl.ds(base, R)], am_v)

    @pl.loop(0, R // L)
    def _mask(j):
        s = pl.ds(j * L, L)
        am_v[s] = (am_v[s] - 1.0) * 10000.0

    pltpu.sync_copy(am_v, mask_hbm.at[pl.ds(base, R)])

    @pl.loop(0, NCH, step=2)
    def _ring(i):
        for b in range(2):
            g = i + b
            nb = 1 - b

            if b == 0:
                @pl.when(i > 0)
                def _():
                    wait_out(nb)
            else:
                wait_out(nb)

            @pl.when(g + 1 < NCH)
            def _():
                start_gather(g + 1, nb)

            wait_gather(b)

            @pl.loop(0, C)
            def _row(r):
                for j in range(DL):
                    s = pl.ds(j * L, L)
                    tok_rows[b, r, s] = tok_rows[b, r, s] + pos_rows[b, r, s]

            pltpu.async_copy(
                tok_rows.at[b], out_hbm.at[pl.ds(base + g * C, C)], sem_o[b])

    wait_out(1)


def kernel(input_ids, attention_mask, position_ids, wte, wpe):
    B, S = input_ids.shape
    ids = input_ids.reshape(-1).astype(jnp.int32)
    pos = position_ids.reshape(-1).astype(jnp.int32)
    am = attention_mask.reshape(-1)
    hidden, mask = _emb_kernel(ids, pos, am, wte, wpe)
    return (hidden.reshape(B, S, D), mask.reshape(1, 1, B, S))


# speedup vs baseline: 1.1556x; 1.0832x over previous
import functools

import jax
import jax.numpy as jnp
from jax import lax
from jax.experimental import pallas as pl
from jax.experimental.pallas import tpu as pltpu
from jax.experimental.pallas import tpu_sc as plsc

NC = 2
NS = 16
L = 16
NW = NC * NS

TOKENS = 8192
D = 768
R = TOKENS // NW
C = 64
NCH = R // C
DL = D // L

_mesh = plsc.VectorSubcoreMesh(core_axis_name="c", subcore_axis_name="s")


@functools.partial(
    pl.kernel,
    out_type=(
        jax.ShapeDtypeStruct((TOKENS, D), jnp.float32),
        jax.ShapeDtypeStruct((TOKENS,), jnp.float32),
    ),
    mesh=_mesh,
    scratch_types=[
        pltpu.VMEM((C,), jnp.int32),
        pltpu.VMEM((C,), jnp.int32),
        pltpu.VMEM((C, D), jnp.float32),
        pltpu.VMEM((C, D), jnp.float32),
        pltpu.VMEM((R,), jnp.float32),
        pltpu.SemaphoreType.DMA,
    ],
)
def _emb_kernel(ids_hbm, pos_hbm, am_hbm, wte_hbm, wpe_hbm, out_hbm, mask_hbm,
                tok_idx, pos_idx, tok_rows, pos_rows, am_v, sem):
    wid = lax.axis_index("s") * NC + lax.axis_index("c")
    base = wid * R

    pltpu.sync_copy(am_hbm.at[pl.ds(base, R)], am_v)

    @pl.loop(0, R // L)
    def _mask(j):
        s = pl.ds(j * L, L)
        am_v[s] = (am_v[s] - 1.0) * 10000.0

    pltpu.sync_copy(am_v, mask_hbm.at[pl.ds(base, R)])

    @pl.loop(0, NCH)
    def _chunk(i):
        off = base + i * C
        pltpu.sync_copy(ids_hbm.at[pl.ds(off, C)], tok_idx)
        pltpu.sync_copy(pos_hbm.at[pl.ds(off, C)], pos_idx)
        h1 = pltpu.async_copy(wte_hbm.at[tok_idx], tok_rows, sem)
        h2 = pltpu.async_copy(wpe_hbm.at[pos_idx], pos_rows, sem)
        h1.wait()
        h2.wait()

        @pl.loop(0, C)
        def _row(r):
            for j in range(DL):
                s = pl.ds(j * L, L)
                tok_rows[r, s] = tok_rows[r, s] + pos_rows[r, s]

        pltpu.sync_copy(tok_rows, out_hbm.at[pl.ds(off, C)])


def kernel(input_ids, attention_mask, position_ids, wte, wpe):
    B, S = input_ids.shape
    ids = input_ids.reshape(-1).astype(jnp.int32)
    pos = position_ids.reshape(-1).astype(jnp.int32)
    am = attention_mask.reshape(-1)
    hidden, mask = _emb_kernel(ids, pos, am, wte, wpe)
    return (hidden.reshape(B, S, D), mask.reshape(1, 1, B, S))
